# baseline (speedup 1.0000x reference)
import functools

import jax
import jax.numpy as jnp
from jax import lax
from jax.experimental import pallas as pl
from jax.experimental.pallas import tpu as pltpu
from jax.experimental.pallas import tpu_sc as plsc

NUM_CORES = 2
NUM_SUBCORES = 16
NUM_WORKERS = NUM_CORES * NUM_SUBCORES
LANES = 16
CHUNK = 1024


def _make_deg_kernel(np_, e, epw):
    rows_per_tile = np_ // NUM_SUBCORES
    vregs_per_tile = rows_per_tile // LANES
    fc = epw // CHUNK
    tail = epw - fc * CHUNK

    mesh = plsc.VectorSubcoreMesh(
        core_axis_name="c", subcore_axis_name="s",
        num_cores=NUM_CORES, num_subcores=NUM_SUBCORES)

    def body(ei_hbm, cnt_hbm, deg_sh, idx_v, ones_v, deg_v, cnt_v, tidx_v,
             isem, ssem):
        c = lax.axis_index("c")
        s = lax.axis_index("s")
        w = s * NUM_CORES + c
        ebase = w * epw

        def zb(i, carry):
            deg_v[pl.ds(i * LANES, LANES)] = jnp.zeros((LANES,), jnp.float32)
            return carry
        lax.fori_loop(0, vregs_per_tile, zb, None)
        pltpu.sync_copy(deg_v, deg_sh.at[pl.ds(s * rows_per_tile, rows_per_tile)])

        def ob(i, carry):
            ones_v[pl.ds(i * LANES, LANES)] = jnp.ones((LANES,), jnp.float32)
            return carry
        lax.fori_loop(0, CHUNK // LANES, ob, None)
        pltpu.sync_copy(ei_hbm.at[1].at[pl.ds(ebase, epw)], idx_v)

        plsc.subcore_barrier()

        def fire(j, carry):
            pltpu.async_copy(ones_v, deg_sh.at[idx_v.at[pl.ds(j * CHUNK, CHUNK)]],
                             ssem, add=True)
            return carry
        lax.fori_loop(0, fc, fire, None)

        def drain(j, carry):
            pltpu.make_async_copy(ones_v, deg_sh.at[idx_v.at[pl.ds(0, CHUNK)]],
                                  ssem).wait()
            return carry
        lax.fori_loop(0, fc, drain, None)

        if tail:
            pltpu.sync_copy(ei_hbm.at[1].at[pl.ds(ebase + fc * CHUNK, tail)],
                            tidx_v)
            pltpu.sync_copy(ones_v.at[pl.ds(0, tail)], deg_sh.at[tidx_v],
                            add=True)

        plsc.subcore_barrier()

        base = s * rows_per_tile
        pltpu.sync_copy(deg_sh.at[pl.ds(base, rows_per_tile)], deg_v)

        def bc(i, carry):
            v = deg_v[pl.ds(i * LANES, LANES)]
            for l in range(LANES):
                cnt_v[i * LANES + l, :] = jnp.full((LANES,), v[l], jnp.float32)
            return carry
        lax.fori_loop(0, vregs_per_tile, bc, None)
        pltpu.sync_copy(cnt_v, cnt_hbm.at[c].at[pl.ds(base, rows_per_tile)])

    return pl.kernel(
        body,
        out_type=jax.ShapeDtypeStruct((NUM_CORES, np_, LANES), jnp.float32),
        mesh=mesh,
        compiler_params=pltpu.CompilerParams(use_tc_tiling_on_sc=False),
        scratch_types=[
            pltpu.VMEM_SHARED((np_,), jnp.float32),
            pltpu.VMEM((epw,), jnp.int32),
            pltpu.VMEM((CHUNK,), jnp.float32),
            pltpu.VMEM((rows_per_tile,), jnp.float32),
            pltpu.VMEM((rows_per_tile, LANES), jnp.float32),
            pltpu.VMEM((tail if tail else LANES,), jnp.int32),
            pltpu.SemaphoreType.DMA,
            pltpu.SemaphoreType.DMA,
        ],
    )


def _make_agg_kernel(np_, e, epw):
    rows_per_tile = np_ // NUM_SUBCORES
    fc = epw // CHUNK
    tail = epw - fc * CHUNK

    mesh = plsc.VectorSubcoreMesh(
        core_axis_name="c", subcore_axis_name="s",
        num_cores=NUM_CORES, num_subcores=NUM_SUBCORES)

    def body(g_hbm, ei_hbm, out_hbm, acc_sh, srcb, dstb,
             rows0, rows1, rows2, rows3, zt, tidx_v, trows, gsem, ssem):
        c = lax.axis_index("c")
        s = lax.axis_index("s")
        w = s * NUM_CORES + c
        ebase = w * epw

        def zb(i, carry):
            zt[i, :] = jnp.zeros((LANES,), jnp.float32)
            return carry
        lax.fori_loop(0, rows_per_tile, zb, None)
        pltpu.sync_copy(zt, acc_sh.at[pl.ds(s * rows_per_tile, rows_per_tile)])
        pltpu.sync_copy(ei_hbm.at[0].at[pl.ds(ebase, epw)], srcb)
        pltpu.sync_copy(ei_hbm.at[1].at[pl.ds(ebase, epw)], dstb)
        plsc.subcore_barrier()

        rows = (rows0, rows1, rows2, rows3)
        nbuf = len(rows)
        depth = nbuf - 1

        def sidx(j):
            return srcb.at[pl.ds(j * CHUNK, CHUNK)]

        def didx(j):
            return dstb.at[pl.ds(j * CHUNK, CHUNK)]

        for k in range(depth):
            pltpu.async_copy(g_hbm.at[sidx(k)], rows[k], gsem)

        def group(jg, carry):
            for b in range(nbuf):
                j = jg * nbuf + b
                buf = rows[b]
                pltpu.make_async_copy(g_hbm.at[sidx(j)], buf, gsem).wait()
                pltpu.async_copy(buf, acc_sh.at[didx(j)], ssem, add=True)
                prv = rows[(b - 1) % nbuf]

                @pl.when(j >= 1)
                def _():
                    pltpu.make_async_copy(prv, acc_sh.at[didx(j - 1)],
                                          ssem).wait()

                @pl.when(j + depth < fc)
                def _():
                    pltpu.async_copy(g_hbm.at[sidx(j + depth)], prv, gsem)
            return carry
        lax.fori_loop(0, fc // nbuf, group, None)
        for j in range(fc - fc % nbuf, fc):
            b = j % nbuf
            pltpu.make_async_copy(g_hbm.at[sidx(j)], rows[b], gsem).wait()
            pltpu.async_copy(rows[b], acc_sh.at[didx(j)], ssem, add=True)
            pltpu.make_async_copy(rows[(b - 1) % nbuf],
                                  acc_sh.at[didx(j - 1)], ssem).wait()
        pltpu.make_async_copy(rows[(fc - 1) % nbuf],
                              acc_sh.at[didx(fc - 1)], ssem).wait()

        if tail:
            pltpu.sync_copy(ei_hbm.at[0].at[pl.ds(ebase + fc * CHUNK, tail)],
                            tidx_v)
            pltpu.async_copy(g_hbm.at[tidx_v], trows, gsem).wait()
            pltpu.sync_copy(ei_hbm.at[1].at[pl.ds(ebase + fc * CHUNK, tail)],
                            tidx_v)
            pltpu.sync_copy(trows, acc_sh.at[tidx_v], add=True)

        plsc.subcore_barrier()
        base = s * rows_per_tile
        pltpu.sync_copy(acc_sh.at[pl.ds(base, rows_per_tile)],
                        out_hbm.at[c].at[pl.ds(base, rows_per_tile)])

    return pl.kernel(
        body,
        out_type=jax.ShapeDtypeStruct((NUM_CORES, np_, LANES), jnp.float32),
        mesh=mesh,
        compiler_params=pltpu.CompilerParams(use_tc_tiling_on_sc=False),
        scratch_types=[
            pltpu.VMEM_SHARED((np_, LANES), jnp.float32),
            pltpu.VMEM((epw,), jnp.int32),
            pltpu.VMEM((epw,), jnp.int32),
            pltpu.VMEM((CHUNK, LANES), jnp.float32),
            pltpu.VMEM((CHUNK, LANES), jnp.float32),
            pltpu.VMEM((CHUNK, LANES), jnp.float32),
            pltpu.VMEM((CHUNK, LANES), jnp.float32),
            pltpu.VMEM((rows_per_tile, LANES), jnp.float32),
            pltpu.VMEM((tail if tail else LANES,), jnp.int32),
            pltpu.VMEM((tail if tail else LANES, LANES), jnp.float32),
            pltpu.SemaphoreType.DMA,
            pltpu.SemaphoreType.DMA,
        ],
    )


_FW = 128
_GRP = _FW // LANES


def _tc_scale_matmul(x8, w1b, cnt_f, nf, npf):
    def body(x_ref, w_ref, c_ref, o_ref, d_ref):
        dis = jax.lax.rsqrt(c_ref[0, :nf] + c_ref[1, :nf] + 1.0)
        d_ref[...] = dis
        h = jnp.dot(x_ref[...], w_ref[...], preferred_element_type=jnp.float32)
        o_ref[...] = dis * h

    return pl.pallas_call(
        body,
        out_shape=[jax.ShapeDtypeStruct((nf, _FW), jnp.float32),
                   jax.ShapeDtypeStruct((nf, _FW), jnp.float32)],
    )(x8, w1b, cnt_f)


def _tc_mid_layer(p_f, g1_f, dis_f, b1, w2b, nf):
    def body(p_ref, g_ref, d_ref, b_ref, w_ref, o_ref):
        agg = p_ref[0, :nf] + p_ref[1, :nf] + g_ref[...]
        h = jnp.maximum(d_ref[...] * agg + b_ref[...], 0.0)
        o_ref[...] = d_ref[...] * jnp.dot(
            h, w_ref[...], preferred_element_type=jnp.float32)

    return pl.pallas_call(
        body,
        out_shape=jax.ShapeDtypeStruct((nf, _FW), jnp.float32),
    )(p_f, g1_f, dis_f, b1, w2b)


def _tc_final_layer(q_f, g2_f, dis_f, b2, nf):
    def body(q_ref, g_ref, d_ref, b_ref, o_ref):
        agg = q_ref[0, :nf] + q_ref[1, :nf] + g_ref[...]
        o_ref[...] = jax.nn.sigmoid(d_ref[...] * agg + b_ref[...])

    return pl.pallas_call(
        body,
        out_shape=jax.ShapeDtypeStruct((nf, _FW), jnp.float32),
    )(q_f, g2_f, dis_f, b2)


def kernel(x, edge_index, W1, b1, W2, b2):
    n, f0 = x.shape
    e = edge_index.shape[1]
    f1 = W1.shape[1]
    f2 = W2.shape[1]

    np_ = -(-(n + LANES) // 256) * 256
    if e % (NUM_WORKERS * LANES):
        e_pad = -(-e // (NUM_WORKERS * LANES)) * NUM_WORKERS * LANES
        padi = jnp.arange(e_pad - e, dtype=edge_index.dtype)
        ei = jnp.concatenate(
            [edge_index, jnp.stack([padi % LANES, n + (padi % LANES)])], axis=1)
    else:
        e_pad = e
        ei = edge_index
    epw = e_pad // NUM_WORKERS
    nf = n * LANES // _FW
    npf = np_ * LANES // _FW

    b1row = jnp.tile(b1, _GRP).reshape(1, _FW)
    w1b = jnp.kron(jnp.eye(_GRP, dtype=jnp.float32), W1)
    w2pad = jnp.zeros((f1, LANES), jnp.float32).at[:, :f2].set(W2)
    w2b = jnp.kron(jnp.eye(_GRP, dtype=jnp.float32), w2pad)
    b2row = jnp.tile(jnp.zeros((LANES,), jnp.float32).at[:f2].set(b2),
                     _GRP).reshape(1, _FW)

    cnt = _make_deg_kernel(np_, e_pad, epw)(ei)
    agg = _make_agg_kernel(np_, e_pad, epw)

    g1_f, dis_f = _tc_scale_matmul(x.reshape(nf, f0 * _GRP), w1b,
                                   cnt.reshape(2, npf, _FW), nf, npf)
    p = agg(g1_f.reshape(n, LANES), ei)
    g2_f = _tc_mid_layer(p.reshape(2, npf, _FW), g1_f, dis_f, b1row, w2b, nf)
    q = agg(g2_f.reshape(n, LANES), ei)
    out_f = _tc_final_layer(q.reshape(2, npf, _FW), g2_f, dis_f, b2row, nf)
    return out_f.reshape(n, LANES)[:, :f2]


# speedup vs baseline: 1.0058x; 1.0058x over previous
import functools

import jax
import jax.numpy as jnp
from jax import lax
from jax.experimental import pallas as pl
from jax.experimental.pallas import tpu as pltpu
from jax.experimental.pallas import tpu_sc as plsc

NUM_CORES = 2
NUM_SUBCORES = 16
NUM_WORKERS = NUM_CORES * NUM_SUBCORES
LANES = 16
CHUNK = 512


def _make_deg_kernel(np_, e, epw):
    rows_per_tile = np_ // NUM_SUBCORES
    vregs_per_tile = rows_per_tile // LANES
    fc = epw // CHUNK
    tail = epw - fc * CHUNK

    mesh = plsc.VectorSubcoreMesh(
        core_axis_name="c", subcore_axis_name="s",
        num_cores=NUM_CORES, num_subcores=NUM_SUBCORES)

    def body(ei_hbm, cnt_hbm, deg_sh, idx_v, ones_v, deg_v, cnt_v, tidx_v,
             isem, ssem):
        c = lax.axis_index("c")
        s = lax.axis_index("s")
        w = s * NUM_CORES + c
        ebase = w * epw

        def zb(i, carry):
            deg_v[pl.ds(i * LANES, LANES)] = jnp.zeros((LANES,), jnp.float32)
            return carry
        lax.fori_loop(0, vregs_per_tile, zb, None)
        pltpu.sync_copy(deg_v, deg_sh.at[pl.ds(s * rows_per_tile, rows_per_tile)])

        def ob(i, carry):
            ones_v[pl.ds(i * LANES, LANES)] = jnp.ones((LANES,), jnp.float32)
            return carry
        lax.fori_loop(0, CHUNK // LANES, ob, None)
        pltpu.sync_copy(ei_hbm.at[1].at[pl.ds(ebase, epw)], idx_v)

        plsc.subcore_barrier()

        def fire(j, carry):
            pltpu.async_copy(ones_v, deg_sh.at[idx_v.at[pl.ds(j * CHUNK, CHUNK)]],
                             ssem, add=True)
            return carry
        lax.fori_loop(0, fc, fire, None)

        def drain(j, carry):
            pltpu.make_async_copy(ones_v, deg_sh.at[idx_v.at[pl.ds(0, CHUNK)]],
                                  ssem).wait()
            return carry
        lax.fori_loop(0, fc, drain, None)

        if tail:
            pltpu.sync_copy(ei_hbm.at[1].at[pl.ds(ebase + fc * CHUNK, tail)],
                            tidx_v)
            pltpu.sync_copy(ones_v.at[pl.ds(0, tail)], deg_sh.at[tidx_v],
                            add=True)

        plsc.subcore_barrier()

        base = s * rows_per_tile
        pltpu.sync_copy(deg_sh.at[pl.ds(base, rows_per_tile)], deg_v)

        def bc(i, carry):
            v = deg_v[pl.ds(i * LANES, LANES)]
            for l in range(LANES):
                cnt_v[i * LANES + l, :] = jnp.full((LANES,), v[l], jnp.float32)
            return carry
        lax.fori_loop(0, vregs_per_tile, bc, None)
        pltpu.sync_copy(cnt_v, cnt_hbm.at[c].at[pl.ds(base, rows_per_tile)])

    return pl.kernel(
        body,
        out_type=jax.ShapeDtypeStruct((NUM_CORES, np_, LANES), jnp.float32),
        mesh=mesh,
        compiler_params=pltpu.CompilerParams(use_tc_tiling_on_sc=False),
        scratch_types=[
            pltpu.VMEM_SHARED((np_,), jnp.float32),
            pltpu.VMEM((epw,), jnp.int32),
            pltpu.VMEM((CHUNK,), jnp.float32),
            pltpu.VMEM((rows_per_tile,), jnp.float32),
            pltpu.VMEM((rows_per_tile, LANES), jnp.float32),
            pltpu.VMEM((tail if tail else LANES,), jnp.int32),
            pltpu.SemaphoreType.DMA,
            pltpu.SemaphoreType.DMA,
        ],
    )


def _make_agg_kernel(np_, e, epw):
    rows_per_tile = np_ // NUM_SUBCORES
    fc = epw // CHUNK
    tail = epw - fc * CHUNK

    mesh = plsc.VectorSubcoreMesh(
        core_axis_name="c", subcore_axis_name="s",
        num_cores=NUM_CORES, num_subcores=NUM_SUBCORES)

    def body(g_hbm, ei_hbm, out_hbm, acc_sh, srcb, dstb,
             rows0, rows1, rows2, rows3, zt, tidx_v, trows, gsem, ssem):
        c = lax.axis_index("c")
        s = lax.axis_index("s")
        w = s * NUM_CORES + c
        ebase = w * epw

        def zb(i, carry):
            zt[i, :] = jnp.zeros((LANES,), jnp.float32)
            return carry
        lax.fori_loop(0, rows_per_tile, zb, None)
        pltpu.sync_copy(zt, acc_sh.at[pl.ds(s * rows_per_tile, rows_per_tile)])
        pltpu.sync_copy(ei_hbm.at[0].at[pl.ds(ebase, epw)], srcb)
        pltpu.sync_copy(ei_hbm.at[1].at[pl.ds(ebase, epw)], dstb)
        plsc.subcore_barrier()

        rows = (rows0, rows1, rows2, rows3)
        nbuf = len(rows)
        depth = nbuf - 1

        def sidx(j):
            return srcb.at[pl.ds(j * CHUNK, CHUNK)]

        def didx(j):
            return dstb.at[pl.ds(j * CHUNK, CHUNK)]

        for k in range(depth):
            pltpu.async_copy(g_hbm.at[sidx(k)], rows[k], gsem)

        def group(jg, carry):
            for b in range(nbuf):
                j = jg * nbuf + b
                buf = rows[b]
                pltpu.make_async_copy(g_hbm.at[sidx(j)], buf, gsem).wait()
                pltpu.async_copy(buf, acc_sh.at[didx(j)], ssem, add=True)
                prv = rows[(b - 1) % nbuf]

                @pl.when(j >= 1)
                def _():
                    pltpu.make_async_copy(prv, acc_sh.at[didx(j - 1)],
                                          ssem).wait()

                @pl.when(j + depth < fc)
                def _():
                    pltpu.async_copy(g_hbm.at[sidx(j + depth)], prv, gsem)
            return carry
        lax.fori_loop(0, fc // nbuf, group, None)
        for j in range(fc - fc % nbuf, fc):
            b = j % nbuf
            pltpu.make_async_copy(g_hbm.at[sidx(j)], rows[b], gsem).wait()
            pltpu.async_copy(rows[b], acc_sh.at[didx(j)], ssem, add=True)
            pltpu.make_async_copy(rows[(b - 1) % nbuf],
                                  acc_sh.at[didx(j - 1)], ssem).wait()
        pltpu.make_async_copy(rows[(fc - 1) % nbuf],
                              acc_sh.at[didx(fc - 1)], ssem).wait()

        if tail:
            pltpu.sync_copy(ei_hbm.at[0].at[pl.ds(ebase + fc * CHUNK, tail)],
                            tidx_v)
            pltpu.async_copy(g_hbm.at[tidx_v], trows, gsem).wait()
            pltpu.sync_copy(ei_hbm.at[1].at[pl.ds(ebase + fc * CHUNK, tail)],
                            tidx_v)
            pltpu.sync_copy(trows, acc_sh.at[tidx_v], add=True)

        plsc.subcore_barrier()
        base = s * rows_per_tile
        pltpu.sync_copy(acc_sh.at[pl.ds(base, rows_per_tile)],
                        out_hbm.at[c].at[pl.ds(base, rows_per_tile)])

    return pl.kernel(
        body,
        out_type=jax.ShapeDtypeStruct((NUM_CORES, np_, LANES), jnp.float32),
        mesh=mesh,
        compiler_params=pltpu.CompilerParams(use_tc_tiling_on_sc=False),
        scratch_types=[
            pltpu.VMEM_SHARED((np_, LANES), jnp.float32),
            pltpu.VMEM((epw,), jnp.int32),
            pltpu.VMEM((epw,), jnp.int32),
            pltpu.VMEM((CHUNK, LANES), jnp.float32),
            pltpu.VMEM((CHUNK, LANES), jnp.float32),
            pltpu.VMEM((CHUNK, LANES), jnp.float32),
            pltpu.VMEM((CHUNK, LANES), jnp.float32),
            pltpu.VMEM((rows_per_tile, LANES), jnp.float32),
            pltpu.VMEM((tail if tail else LANES,), jnp.int32),
            pltpu.VMEM((tail if tail else LANES, LANES), jnp.float32),
            pltpu.SemaphoreType.DMA,
            pltpu.SemaphoreType.DMA,
        ],
    )


_FW = 128
_GRP = _FW // LANES


def _tc_scale_matmul(x8, w1b, cnt_f, nf, npf):
    def body(x_ref, w_ref, c_ref, o_ref, d_ref):
        dis = jax.lax.rsqrt(c_ref[0, :nf] + c_ref[1, :nf] + 1.0)
        d_ref[...] = dis
        h = jnp.dot(x_ref[...], w_ref[...], preferred_element_type=jnp.float32)
        o_ref[...] = dis * h

    return pl.pallas_call(
        body,
        out_shape=[jax.ShapeDtypeStruct((nf, _FW), jnp.float32),
                   jax.ShapeDtypeStruct((nf, _FW), jnp.float32)],
    )(x8, w1b, cnt_f)


def _tc_mid_layer(p_f, g1_f, dis_f, b1, w2b, nf):
    def body(p_ref, g_ref, d_ref, b_ref, w_ref, o_ref):
        agg = p_ref[0, :nf] + p_ref[1, :nf] + g_ref[...]
        h = jnp.maximum(d_ref[...] * agg + b_ref[...], 0.0)
        o_ref[...] = d_ref[...] * jnp.dot(
            h, w_ref[...], preferred_element_type=jnp.float32)

    return pl.pallas_call(
        body,
        out_shape=jax.ShapeDtypeStruct((nf, _FW), jnp.float32),
    )(p_f, g1_f, dis_f, b1, w2b)


def _tc_final_layer(q_f, g2_f, dis_f, b2, nf):
    def body(q_ref, g_ref, d_ref, b_ref, o_ref):
        agg = q_ref[0, :nf] + q_ref[1, :nf] + g_ref[...]
        o_ref[...] = jax.nn.sigmoid(d_ref[...] * agg + b_ref[...])

    return pl.pallas_call(
        body,
        out_shape=jax.ShapeDtypeStruct((nf, _FW), jnp.float32),
    )(q_f, g2_f, dis_f, b2)


def kernel(x, edge_index, W1, b1, W2, b2):
    n, f0 = x.shape
    e = edge_index.shape[1]
    f1 = W1.shape[1]
    f2 = W2.shape[1]

    np_ = -(-(n + LANES) // 256) * 256
    if e % (NUM_WORKERS * LANES):
        e_pad = -(-e // (NUM_WORKERS * LANES)) * NUM_WORKERS * LANES
        padi = jnp.arange(e_pad - e, dtype=edge_index.dtype)
        ei = jnp.concatenate(
            [edge_index, jnp.stack([padi % LANES, n + (padi % LANES)])], axis=1)
    else:
        e_pad = e
        ei = edge_index
    epw = e_pad // NUM_WORKERS
    nf = n * LANES // _FW
    npf = np_ * LANES // _FW

    b1row = jnp.tile(b1, _GRP).reshape(1, _FW)
    w1b = jnp.kron(jnp.eye(_GRP, dtype=jnp.float32), W1)
    w2pad = jnp.zeros((f1, LANES), jnp.float32).at[:, :f2].set(W2)
    w2b = jnp.kron(jnp.eye(_GRP, dtype=jnp.float32), w2pad)
    b2row = jnp.tile(jnp.zeros((LANES,), jnp.float32).at[:f2].set(b2),
                     _GRP).reshape(1, _FW)

    cnt = _make_deg_kernel(np_, e_pad, epw)(ei)
    agg = _make_agg_kernel(np_, e_pad, epw)

    g1_f, dis_f = _tc_scale_matmul(x.reshape(nf, f0 * _GRP), w1b,
                                   cnt.reshape(2, npf, _FW), nf, npf)
    p = agg(g1_f.reshape(n, LANES), ei)
    g2_f = _tc_mid_layer(p.reshape(2, npf, _FW), g1_f, dis_f, b1row, w2b, nf)
    q = agg(g2_f.reshape(n, LANES), ei)
    out_f = _tc_final_layer(q.reshape(2, npf, _FW), g2_f, dis_f, b2row, nf)
    return out_f.reshape(n, LANES)[:, :f2]


# speedup vs baseline: 1.0411x; 1.0351x over previous
import functools

import jax
import jax.numpy as jnp
from jax import lax
from jax.experimental import pallas as pl
from jax.experimental.pallas import tpu as pltpu
from jax.experimental.pallas import tpu_sc as plsc

NUM_CORES = 2
NUM_SUBCORES = 16
NUM_WORKERS = NUM_CORES * NUM_SUBCORES
LANES = 16
CHUNK = 512


def _make_deg_kernel(np_, e, epw):
    rows_per_tile = np_ // NUM_SUBCORES
    vregs_per_tile = rows_per_tile // LANES
    fc = epw // CHUNK
    tail = epw - fc * CHUNK

    mesh = plsc.VectorSubcoreMesh(
        core_axis_name="c", subcore_axis_name="s",
        num_cores=NUM_CORES, num_subcores=NUM_SUBCORES)

    def body(ei_hbm, cnt_hbm, deg_sh, idx_v, ones_v, deg_v, cnt_v, tidx_v,
             isem, ssem):
        c = lax.axis_index("c")
        s = lax.axis_index("s")
        w = s * NUM_CORES + c
        ebase = w * epw

        def zb(i, carry):
            deg_v[pl.ds(i * LANES, LANES)] = jnp.zeros((LANES,), jnp.float32)
            return carry
        lax.fori_loop(0, vregs_per_tile, zb, None)
        pltpu.sync_copy(deg_v, deg_sh.at[pl.ds(s * rows_per_tile, rows_per_tile)])

        def ob(i, carry):
            ones_v[pl.ds(i * LANES, LANES)] = jnp.ones((LANES,), jnp.float32)
            return carry
        lax.fori_loop(0, CHUNK // LANES, ob, None)
        pltpu.sync_copy(ei_hbm.at[1].at[pl.ds(ebase, epw)], idx_v)

        plsc.subcore_barrier()

        def fire(j, carry):
            pltpu.async_copy(ones_v, deg_sh.at[idx_v.at[pl.ds(j * CHUNK, CHUNK)]],
                             ssem, add=True)
            return carry
        lax.fori_loop(0, fc, fire, None)

        def drain(j, carry):
            pltpu.make_async_copy(ones_v, deg_sh.at[idx_v.at[pl.ds(0, CHUNK)]],
                                  ssem).wait()
            return carry
        lax.fori_loop(0, fc, drain, None)

        if tail:
            pltpu.sync_copy(ei_hbm.at[1].at[pl.ds(ebase + fc * CHUNK, tail)],
                            tidx_v)
            pltpu.sync_copy(ones_v.at[pl.ds(0, tail)], deg_sh.at[tidx_v],
                            add=True)

        plsc.subcore_barrier()

        base = s * rows_per_tile
        pltpu.sync_copy(deg_sh.at[pl.ds(base, rows_per_tile)], deg_v)

        def bc(i, carry):
            v = deg_v[pl.ds(i * LANES, LANES)]
            for l in range(LANES):
                cnt_v[i * LANES + l, :] = jnp.full((LANES,), v[l], jnp.float32)
            return carry
        lax.fori_loop(0, vregs_per_tile, bc, None)
        pltpu.sync_copy(cnt_v, cnt_hbm.at[c].at[pl.ds(base, rows_per_tile)])

    return pl.kernel(
        body,
        out_type=jax.ShapeDtypeStruct((NUM_CORES, np_, LANES), jnp.float32),
        mesh=mesh,
        compiler_params=pltpu.CompilerParams(use_tc_tiling_on_sc=False),
        scratch_types=[
            pltpu.VMEM_SHARED((np_,), jnp.float32),
            pltpu.VMEM((epw,), jnp.int32),
            pltpu.VMEM((CHUNK,), jnp.float32),
            pltpu.VMEM((rows_per_tile,), jnp.float32),
            pltpu.VMEM((rows_per_tile, LANES), jnp.float32),
            pltpu.VMEM((tail if tail else LANES,), jnp.int32),
            pltpu.SemaphoreType.DMA,
            pltpu.SemaphoreType.DMA,
        ],
    )


def _make_agg_kernel(np_, e, epw, n):
    rows_per_tile = np_ // NUM_SUBCORES
    fc = epw // CHUNK
    tail = epw - fc * CHUNK
    lt = n // NUM_SUBCORES
    lt = -(-lt // 8) * 8
    nload = -(-n // lt)

    mesh = plsc.VectorSubcoreMesh(
        core_axis_name="c", subcore_axis_name="s",
        num_cores=NUM_CORES, num_subcores=NUM_SUBCORES)

    def body(g_hbm, ei_hbm, out_hbm, acc_sh, g_sh, srcb, dstb,
             rows0, rows1, rows2, rows3, zt, tidx_v, trows, gsem, ssem):
        c = lax.axis_index("c")
        s = lax.axis_index("s")
        w = s * NUM_CORES + c
        ebase = w * epw

        @pl.when(s < nload)
        def _():
            rows_here = min(lt, n - (nload - 1) * lt)
            lbase = s * lt

            @pl.when(s < nload - 1)
            def _():
                pltpu.sync_copy(g_hbm.at[pl.ds(lbase, lt)],
                                g_sh.at[pl.ds(lbase, lt)])

            @pl.when(s == nload - 1)
            def _():
                pltpu.sync_copy(g_hbm.at[pl.ds((nload - 1) * lt, rows_here)],
                                g_sh.at[pl.ds((nload - 1) * lt, rows_here)])

        def zb(i, carry):
            zt[i, :] = jnp.zeros((LANES,), jnp.float32)
            return carry
        lax.fori_loop(0, rows_per_tile, zb, None)
        pltpu.sync_copy(zt, acc_sh.at[pl.ds(s * rows_per_tile, rows_per_tile)])
        pltpu.sync_copy(ei_hbm.at[0].at[pl.ds(ebase, epw)], srcb)
        pltpu.sync_copy(ei_hbm.at[1].at[pl.ds(ebase, epw)], dstb)
        plsc.subcore_barrier()
        g_hbm = g_sh

        rows = (rows0, rows1, rows2, rows3)
        nbuf = len(rows)
        depth = nbuf - 1

        def sidx(j):
            return srcb.at[pl.ds(j * CHUNK, CHUNK)]

        def didx(j):
            return dstb.at[pl.ds(j * CHUNK, CHUNK)]

        for k in range(depth):
            pltpu.async_copy(g_hbm.at[sidx(k)], rows[k], gsem)

        def group(jg, carry):
            for b in range(nbuf):
                j = jg * nbuf + b
                buf = rows[b]
                pltpu.make_async_copy(g_hbm.at[sidx(j)], buf, gsem).wait()
                pltpu.async_copy(buf, acc_sh.at[didx(j)], ssem, add=True)
                prv = rows[(b - 1) % nbuf]

                @pl.when(j >= 1)
                def _():
                    pltpu.make_async_copy(prv, acc_sh.at[didx(j - 1)],
                                          ssem).wait()

                @pl.when(j + depth < fc)
                def _():
                    pltpu.async_copy(g_hbm.at[sidx(j + depth)], prv, gsem)
            return carry
        lax.fori_loop(0, fc // nbuf, group, None)
        for j in range(fc - fc % nbuf, fc):
            b = j % nbuf
            pltpu.make_async_copy(g_hbm.at[sidx(j)], rows[b], gsem).wait()
            pltpu.async_copy(rows[b], acc_sh.at[didx(j)], ssem, add=True)
            pltpu.make_async_copy(rows[(b - 1) % nbuf],
                                  acc_sh.at[didx(j - 1)], ssem).wait()
        pltpu.make_async_copy(rows[(fc - 1) % nbuf],
                              acc_sh.at[didx(fc - 1)], ssem).wait()

        if tail:
            pltpu.sync_copy(ei_hbm.at[0].at[pl.ds(ebase + fc * CHUNK, tail)],
                            tidx_v)
            pltpu.async_copy(g_hbm.at[tidx_v], trows, gsem).wait()
            pltpu.sync_copy(ei_hbm.at[1].at[pl.ds(ebase + fc * CHUNK, tail)],
                            tidx_v)
            pltpu.sync_copy(trows, acc_sh.at[tidx_v], add=True)

        plsc.subcore_barrier()
        base = s * rows_per_tile
        pltpu.sync_copy(acc_sh.at[pl.ds(base, rows_per_tile)],
                        out_hbm.at[c].at[pl.ds(base, rows_per_tile)])

    return pl.kernel(
        body,
        out_type=jax.ShapeDtypeStruct((NUM_CORES, np_, LANES), jnp.float32),
        mesh=mesh,
        compiler_params=pltpu.CompilerParams(use_tc_tiling_on_sc=False),
        scratch_types=[
            pltpu.VMEM_SHARED((np_, LANES), jnp.float32),
            pltpu.VMEM_SHARED((n, LANES), jnp.float32),
            pltpu.VMEM((epw,), jnp.int32),
            pltpu.VMEM((epw,), jnp.int32),
            pltpu.VMEM((CHUNK, LANES), jnp.float32),
            pltpu.VMEM((CHUNK, LANES), jnp.float32),
            pltpu.VMEM((CHUNK, LANES), jnp.float32),
            pltpu.VMEM((CHUNK, LANES), jnp.float32),
            pltpu.VMEM((rows_per_tile, LANES), jnp.float32),
            pltpu.VMEM((tail if tail else LANES,), jnp.int32),
            pltpu.VMEM((tail if tail else LANES, LANES), jnp.float32),
            pltpu.SemaphoreType.DMA,
            pltpu.SemaphoreType.DMA,
        ],
    )


_FW = 128
_GRP = _FW // LANES


def _tc_scale_matmul(x8, w1b, cnt_f, nf, npf):
    def body(x_ref, w_ref, c_ref, o_ref, d_ref):
        dis = jax.lax.rsqrt(c_ref[0, :nf] + c_ref[1, :nf] + 1.0)
        d_ref[...] = dis
        h = jnp.dot(x_ref[...], w_ref[...], preferred_element_type=jnp.float32)
        o_ref[...] = dis * h

    return pl.pallas_call(
        body,
        out_shape=[jax.ShapeDtypeStruct((nf, _FW), jnp.float32),
                   jax.ShapeDtypeStruct((nf, _FW), jnp.float32)],
    )(x8, w1b, cnt_f)


def _tc_mid_layer(p_f, g1_f, dis_f, b1, w2b, nf):
    def body(p_ref, g_ref, d_ref, b_ref, w_ref, o_ref):
        agg = p_ref[0, :nf] + p_ref[1, :nf] + g_ref[...]
        h = jnp.maximum(d_ref[...] * agg + b_ref[...], 0.0)
        o_ref[...] = d_ref[...] * jnp.dot(
            h, w_ref[...], preferred_element_type=jnp.float32)

    return pl.pallas_call(
        body,
        out_shape=jax.ShapeDtypeStruct((nf, _FW), jnp.float32),
    )(p_f, g1_f, dis_f, b1, w2b)


def _tc_final_layer(q_f, g2_f, dis_f, b2, nf):
    def body(q_ref, g_ref, d_ref, b_ref, o_ref):
        agg = q_ref[0, :nf] + q_ref[1, :nf] + g_ref[...]
        o_ref[...] = jax.nn.sigmoid(d_ref[...] * agg + b_ref[...])

    return pl.pallas_call(
        body,
        out_shape=jax.ShapeDtypeStruct((nf, _FW), jnp.float32),
    )(q_f, g2_f, dis_f, b2)


def kernel(x, edge_index, W1, b1, W2, b2):
    n, f0 = x.shape
    e = edge_index.shape[1]
    f1 = W1.shape[1]
    f2 = W2.shape[1]

    np_ = -(-(n + LANES) // 256) * 256
    if e % (NUM_WORKERS * LANES):
        e_pad = -(-e // (NUM_WORKERS * LANES)) * NUM_WORKERS * LANES
        padi = jnp.arange(e_pad - e, dtype=edge_index.dtype)
        ei = jnp.concatenate(
            [edge_index, jnp.stack([padi % LANES, n + (padi % LANES)])], axis=1)
    else:
        e_pad = e
        ei = edge_index
    epw = e_pad // NUM_WORKERS
    nf = n * LANES // _FW
    npf = np_ * LANES // _FW

    b1row = jnp.tile(b1, _GRP).reshape(1, _FW)
    w1b = jnp.kron(jnp.eye(_GRP, dtype=jnp.float32), W1)
    w2pad = jnp.zeros((f1, LANES), jnp.float32).at[:, :f2].set(W2)
    w2b = jnp.kron(jnp.eye(_GRP, dtype=jnp.float32), w2pad)
    b2row = jnp.tile(jnp.zeros((LANES,), jnp.float32).at[:f2].set(b2),
                     _GRP).reshape(1, _FW)

    cnt = _make_deg_kernel(np_, e_pad, epw)(ei)
    agg = _make_agg_kernel(np_, e_pad, epw, n)

    g1_f, dis_f = _tc_scale_matmul(x.reshape(nf, f0 * _GRP), w1b,
                                   cnt.reshape(2, npf, _FW), nf, npf)
    p = agg(g1_f.reshape(n, LANES), ei)
    g2_f = _tc_mid_layer(p.reshape(2, npf, _FW), g1_f, dis_f, b1row, w2b, nf)
    q = agg(g2_f.reshape(n, LANES), ei)
    out_f = _tc_final_layer(q.reshape(2, npf, _FW), g2_f, dis_f, b2row, nf)
    return out_f.reshape(n, LANES)[:, :f2]


# speedup vs baseline: 1.0418x; 1.0007x over previous
import functools

import jax
import jax.numpy as jnp
from jax import lax
from jax.experimental import pallas as pl
from jax.experimental.pallas import tpu as pltpu
from jax.experimental.pallas import tpu_sc as plsc

NUM_CORES = 2
NUM_SUBCORES = 16
NUM_WORKERS = NUM_CORES * NUM_SUBCORES
LANES = 16
CHUNK = 512


def _make_deg_kernel(np_, e, epw):
    rows_per_tile = np_ // NUM_SUBCORES
    vregs_per_tile = rows_per_tile // LANES
    fc = epw // CHUNK
    tail = epw - fc * CHUNK

    mesh = plsc.VectorSubcoreMesh(
        core_axis_name="c", subcore_axis_name="s",
        num_cores=NUM_CORES, num_subcores=NUM_SUBCORES)

    def body(ei_hbm, cnt_hbm, deg_sh, idx_v, ones_v, deg_v, cnt_v, tidx_v,
             isem, ssem):
        c = lax.axis_index("c")
        s = lax.axis_index("s")
        w = s * NUM_CORES + c
        ebase = w * epw

        def zb(i, carry):
            deg_v[pl.ds(i * LANES, LANES)] = jnp.zeros((LANES,), jnp.float32)
            return carry
        lax.fori_loop(0, vregs_per_tile, zb, None)
        pltpu.sync_copy(deg_v, deg_sh.at[pl.ds(s * rows_per_tile, rows_per_tile)])

        def ob(i, carry):
            ones_v[pl.ds(i * LANES, LANES)] = jnp.ones((LANES,), jnp.float32)
            return carry
        lax.fori_loop(0, CHUNK // LANES, ob, None)
        pltpu.sync_copy(ei_hbm.at[1].at[pl.ds(ebase, epw)], idx_v)

        plsc.subcore_barrier()

        def fire(j, carry):
            pltpu.async_copy(ones_v, deg_sh.at[idx_v.at[pl.ds(j * CHUNK, CHUNK)]],
                             ssem, add=True)
            return carry
        lax.fori_loop(0, fc, fire, None)

        def drain(j, carry):
            pltpu.make_async_copy(ones_v, deg_sh.at[idx_v.at[pl.ds(0, CHUNK)]],
                                  ssem).wait()
            return carry
        lax.fori_loop(0, fc, drain, None)

        if tail:
            pltpu.sync_copy(ei_hbm.at[1].at[pl.ds(ebase + fc * CHUNK, tail)],
                            tidx_v)
            pltpu.sync_copy(ones_v.at[pl.ds(0, tail)], deg_sh.at[tidx_v],
                            add=True)

        plsc.subcore_barrier()

        base = s * rows_per_tile
        pltpu.sync_copy(deg_sh.at[pl.ds(base, rows_per_tile)], deg_v)

        def bc(i, carry):
            v = deg_v[pl.ds(i * LANES, LANES)]
            for l in range(LANES):
                cnt_v[i * LANES + l, :] = jnp.full((LANES,), v[l], jnp.float32)
            return carry
        lax.fori_loop(0, vregs_per_tile, bc, None)
        pltpu.sync_copy(cnt_v, cnt_hbm.at[c].at[pl.ds(base, rows_per_tile)])

    return pl.kernel(
        body,
        out_type=jax.ShapeDtypeStruct((NUM_CORES, np_, LANES), jnp.float32),
        mesh=mesh,
        compiler_params=pltpu.CompilerParams(use_tc_tiling_on_sc=False),
        scratch_types=[
            pltpu.VMEM_SHARED((np_,), jnp.float32),
            pltpu.VMEM((epw,), jnp.int32),
            pltpu.VMEM((CHUNK,), jnp.float32),
            pltpu.VMEM((rows_per_tile,), jnp.float32),
            pltpu.VMEM((rows_per_tile, LANES), jnp.float32),
            pltpu.VMEM((tail if tail else LANES,), jnp.int32),
            pltpu.SemaphoreType.DMA,
            pltpu.SemaphoreType.DMA,
        ],
    )


def _make_agg_kernel(np_, e, epw, n):
    rows_per_tile = np_ // NUM_SUBCORES
    fc = epw // CHUNK
    tail = epw - fc * CHUNK
    lt = n // NUM_SUBCORES
    lt = -(-lt // 8) * 8
    nload = -(-n // lt)

    mesh = plsc.VectorSubcoreMesh(
        core_axis_name="c", subcore_axis_name="s",
        num_cores=NUM_CORES, num_subcores=NUM_SUBCORES)

    def body(g_hbm, ei_hbm, out_hbm, acc_sh, g_sh, srcb, dstb,
             rows0, rows1, rows2, rows3, zt, tidx_v, trows, gsem, ssem):
        c = lax.axis_index("c")
        s = lax.axis_index("s")
        w = s * NUM_CORES + c
        ebase = w * epw

        @pl.when(s < nload)
        def _():
            rows_here = min(lt, n - (nload - 1) * lt)
            lbase = s * lt

            @pl.when(s < nload - 1)
            def _():
                pltpu.sync_copy(g_hbm.at[pl.ds(lbase, lt)],
                                g_sh.at[pl.ds(lbase, lt)])

            @pl.when(s == nload - 1)
            def _():
                pltpu.sync_copy(g_hbm.at[pl.ds((nload - 1) * lt, rows_here)],
                                g_sh.at[pl.ds((nload - 1) * lt, rows_here)])

        def zb(i, carry):
            zt[i, :] = jnp.zeros((LANES,), jnp.float32)
            return carry
        lax.fori_loop(0, rows_per_tile, zb, None)
        pltpu.sync_copy(zt, acc_sh.at[pl.ds(s * rows_per_tile, rows_per_tile)])
        pltpu.sync_copy(ei_hbm.at[0].at[pl.ds(ebase, epw)], srcb)
        pltpu.sync_copy(ei_hbm.at[1].at[pl.ds(ebase, epw)], dstb)
        plsc.subcore_barrier()
        g_hbm = g_sh

        rows = (rows0, rows1, rows2, rows3)
        nbuf = len(rows)
        depth = nbuf - 1

        def sidx(j):
            return srcb.at[pl.ds(j * CHUNK, CHUNK)]

        def didx(j):
            return dstb.at[pl.ds(j * CHUNK, CHUNK)]

        for k in range(depth):
            pltpu.async_copy(g_hbm.at[sidx(k)], rows[k], gsem)

        def group(jg, carry):
            for b in range(nbuf):
                j = jg * nbuf + b
                buf = rows[b]
                pltpu.make_async_copy(g_hbm.at[sidx(j)], buf, gsem).wait()
                pltpu.async_copy(buf, acc_sh.at[didx(j)], ssem, add=True)
                prv = rows[(b - 1) % nbuf]

                @pl.when(j >= 1)
                def _():
                    pltpu.make_async_copy(prv, acc_sh.at[didx(j - 1)],
                                          ssem).wait()

                @pl.when(j + depth < fc)
                def _():
                    pltpu.async_copy(g_hbm.at[sidx(j + depth)], prv, gsem)
            return carry
        lax.fori_loop(0, fc // nbuf, group, None)
        for j in range(fc - fc % nbuf, fc):
            b = j % nbuf
            pltpu.make_async_copy(g_hbm.at[sidx(j)], rows[b], gsem).wait()
            pltpu.async_copy(rows[b], acc_sh.at[didx(j)], ssem, add=True)
            pltpu.make_async_copy(rows[(b - 1) % nbuf],
                                  acc_sh.at[didx(j - 1)], ssem).wait()
        pltpu.make_async_copy(rows[(fc - 1) % nbuf],
                              acc_sh.at[didx(fc - 1)], ssem).wait()

        if tail:
            pltpu.sync_copy(ei_hbm.at[0].at[pl.ds(ebase + fc * CHUNK, tail)],
                            tidx_v)
            pltpu.async_copy(g_hbm.at[tidx_v], trows, gsem).wait()
            pltpu.sync_copy(ei_hbm.at[1].at[pl.ds(ebase + fc * CHUNK, tail)],
                            tidx_v)
            pltpu.sync_copy(trows, acc_sh.at[tidx_v], add=True)

        plsc.subcore_barrier()
        base = s * rows_per_tile
        pltpu.sync_copy(acc_sh.at[pl.ds(base, rows_per_tile)],
                        out_hbm.at[c].at[pl.ds(base, rows_per_tile)])

    return pl.kernel(
        body,
        out_type=jax.ShapeDtypeStruct((NUM_CORES, np_, LANES), jnp.float32),
        mesh=mesh,
        compiler_params=pltpu.CompilerParams(use_tc_tiling_on_sc=False),
        scratch_types=[
            pltpu.VMEM_SHARED((np_, LANES), jnp.float32),
            pltpu.VMEM_SHARED((n, LANES), jnp.float32),
            pltpu.VMEM((epw,), jnp.int32),
            pltpu.VMEM((epw,), jnp.int32),
            pltpu.VMEM((CHUNK, LANES), jnp.float32),
            pltpu.VMEM((CHUNK, LANES), jnp.float32),
            pltpu.VMEM((CHUNK, LANES), jnp.float32),
            pltpu.VMEM((CHUNK, LANES), jnp.float32),
            pltpu.VMEM((rows_per_tile, LANES), jnp.float32),
            pltpu.VMEM((tail if tail else LANES,), jnp.int32),
            pltpu.VMEM((tail if tail else LANES, LANES), jnp.float32),
            pltpu.SemaphoreType.DMA,
            pltpu.SemaphoreType.DMA,
        ],
    )


_FW = 128
_GRP = _FW // LANES


def _tc_scale_matmul(x8, w1b, cnt_f, nf, npf):
    def body(x_ref, w_ref, c_ref, o_ref, d_ref):
        dis = jax.lax.rsqrt(c_ref[0, :nf] + c_ref[1, :nf] + 1.0)
        d_ref[...] = dis
        h = jnp.dot(x_ref[...], w_ref[...], preferred_element_type=jnp.float32)
        o_ref[...] = dis * h

    return pl.pallas_call(
        body,
        out_shape=[jax.ShapeDtypeStruct((nf, _FW), jnp.float32),
                   jax.ShapeDtypeStruct((nf, _FW), jnp.float32)],
    )(x8, w1b, cnt_f)


def _tc_mid_layer(p_f, g1_f, dis_f, b1, w2b, nf):
    def body(p_ref, g_ref, d_ref, b_ref, w_ref, o_ref):
        agg = p_ref[0, :nf] + p_ref[1, :nf] + g_ref[...]
        h = jnp.maximum(d_ref[...] * agg + b_ref[...], 0.0)
        o_ref[...] = d_ref[...] * jnp.dot(
            h, w_ref[...], preferred_element_type=jnp.float32)

    return pl.pallas_call(
        body,
        out_shape=jax.ShapeDtypeStruct((nf, _FW), jnp.float32),
    )(p_f, g1_f, dis_f, b1, w2b)


def _tc_final_layer(q_f, g2_f, dis_f, b2, sel, nf):
    fw8 = sel.shape[1]

    def body(q_ref, g_ref, d_ref, b_ref, s_ref, o_ref):
        agg = q_ref[0, :nf] + q_ref[1, :nf] + g_ref[...]
        z = jax.nn.sigmoid(d_ref[...] * agg + b_ref[...])
        o_ref[...] = jnp.dot(z, s_ref[...], preferred_element_type=jnp.float32)

    return pl.pallas_call(
        body,
        out_shape=jax.ShapeDtypeStruct((nf, fw8), jnp.float32),
    )(q_f, g2_f, dis_f, b2, sel)


def kernel(x, edge_index, W1, b1, W2, b2):
    n, f0 = x.shape
    e = edge_index.shape[1]
    f1 = W1.shape[1]
    f2 = W2.shape[1]

    np_ = -(-(n + LANES) // 256) * 256
    if e % (NUM_WORKERS * LANES):
        e_pad = -(-e // (NUM_WORKERS * LANES)) * NUM_WORKERS * LANES
        padi = jnp.arange(e_pad - e, dtype=edge_index.dtype)
        ei = jnp.concatenate(
            [edge_index, jnp.stack([padi % LANES, n + (padi % LANES)])], axis=1)
    else:
        e_pad = e
        ei = edge_index
    epw = e_pad // NUM_WORKERS
    nf = n * LANES // _FW
    npf = np_ * LANES // _FW

    b1row = jnp.tile(b1, _GRP).reshape(1, _FW)
    w1b = jnp.kron(jnp.eye(_GRP, dtype=jnp.float32), W1)
    w2pad = jnp.zeros((f1, LANES), jnp.float32).at[:, :f2].set(W2)
    w2b = jnp.kron(jnp.eye(_GRP, dtype=jnp.float32), w2pad)
    b2row = jnp.tile(jnp.zeros((LANES,), jnp.float32).at[:f2].set(b2),
                     _GRP).reshape(1, _FW)
    sel = jnp.kron(jnp.eye(_GRP, dtype=jnp.float32),
                   jnp.eye(LANES, f2, dtype=jnp.float32))

    cnt = _make_deg_kernel(np_, e_pad, epw)(ei)
    agg = _make_agg_kernel(np_, e_pad, epw, n)

    g1_f, dis_f = _tc_scale_matmul(x.reshape(nf, f0 * _GRP), w1b,
                                   cnt.reshape(2, npf, _FW), nf, npf)
    p = agg(g1_f.reshape(n, LANES), ei)
    g2_f = _tc_mid_layer(p.reshape(2, npf, _FW), g1_f, dis_f, b1row, w2b, nf)
    q = agg(g2_f.reshape(n, LANES), ei)
    out_f = _tc_final_layer(q.reshape(2, npf, _FW), g2_f, dis_f, b2row, sel, nf)
    return out_f.reshape(n, f2)


# speedup vs baseline: 1.0431x; 1.0012x over previous
import functools

import jax
import jax.numpy as jnp
from jax import lax
from jax.experimental import pallas as pl
from jax.experimental.pallas import tpu as pltpu
from jax.experimental.pallas import tpu_sc as plsc

NUM_CORES = 2
NUM_SUBCORES = 16
NUM_WORKERS = NUM_CORES * NUM_SUBCORES
LANES = 16
CHUNK = 512


def _make_deg_kernel(np_, e, epw):
    rows_per_tile = np_ // NUM_SUBCORES
    vregs_per_tile = rows_per_tile // LANES
    fc = epw // CHUNK
    tail = epw - fc * CHUNK

    mesh = plsc.VectorSubcoreMesh(
        core_axis_name="c", subcore_axis_name="s",
        num_cores=NUM_CORES, num_subcores=NUM_SUBCORES)

    def body(ei_hbm, cnt_hbm, deg_sh, idx_v, ones_v, deg_v, cnt_v, tidx_v,
             isem, ssem):
        c = lax.axis_index("c")
        s = lax.axis_index("s")
        w = s * NUM_CORES + c
        ebase = w * epw

        def zb(i, carry):
            deg_v[pl.ds(i * LANES, LANES)] = jnp.zeros((LANES,), jnp.float32)
            return carry
        lax.fori_loop(0, vregs_per_tile, zb, None)
        pltpu.sync_copy(deg_v, deg_sh.at[pl.ds(s * rows_per_tile, rows_per_tile)])

        def ob(i, carry):
            ones_v[pl.ds(i * LANES, LANES)] = jnp.ones((LANES,), jnp.float32)
            return carry
        lax.fori_loop(0, CHUNK // LANES, ob, None)
        pltpu.sync_copy(ei_hbm.at[1].at[pl.ds(ebase, epw)], idx_v)

        plsc.subcore_barrier()

        def fire(j, carry):
            pltpu.async_copy(ones_v, deg_sh.at[idx_v.at[pl.ds(j * CHUNK, CHUNK)]],
                             ssem, add=True)
            return carry
        lax.fori_loop(0, fc, fire, None)

        def drain(j, carry):
            pltpu.make_async_copy(ones_v, deg_sh.at[idx_v.at[pl.ds(0, CHUNK)]],
                                  ssem).wait()
            return carry
        lax.fori_loop(0, fc, drain, None)

        if tail:
            pltpu.sync_copy(ei_hbm.at[1].at[pl.ds(ebase + fc * CHUNK, tail)],
                            tidx_v)
            pltpu.sync_copy(ones_v.at[pl.ds(0, tail)], deg_sh.at[tidx_v],
                            add=True)

        plsc.subcore_barrier()

        base = s * rows_per_tile
        pltpu.sync_copy(deg_sh.at[pl.ds(base, rows_per_tile)], deg_v)

        def bc(i, carry):
            v = deg_v[pl.ds(i * LANES, LANES)]
            for l in range(LANES):
                cnt_v[i * LANES + l, :] = jnp.full((LANES,), v[l], jnp.float32)
            return carry
        lax.fori_loop(0, vregs_per_tile, bc, None)
        pltpu.sync_copy(cnt_v, cnt_hbm.at[c].at[pl.ds(base, rows_per_tile)])

    return pl.kernel(
        body,
        out_type=jax.ShapeDtypeStruct((NUM_CORES, np_, LANES), jnp.float32),
        mesh=mesh,
        compiler_params=pltpu.CompilerParams(use_tc_tiling_on_sc=False),
        scratch_types=[
            pltpu.VMEM_SHARED((np_,), jnp.float32),
            pltpu.VMEM((epw,), jnp.int32),
            pltpu.VMEM((CHUNK,), jnp.float32),
            pltpu.VMEM((rows_per_tile,), jnp.float32),
            pltpu.VMEM((rows_per_tile, LANES), jnp.float32),
            pltpu.VMEM((tail if tail else LANES,), jnp.int32),
            pltpu.SemaphoreType.DMA,
            pltpu.SemaphoreType.DMA,
        ],
    )


def _make_agg_kernel(np_, e, epw, n):
    rows_per_tile = np_ // NUM_SUBCORES
    fc = epw // CHUNK
    tail = epw - fc * CHUNK
    lt = n // NUM_SUBCORES
    lt = -(-lt // 8) * 8
    nload = -(-n // lt)

    mesh = plsc.VectorSubcoreMesh(
        core_axis_name="c", subcore_axis_name="s",
        num_cores=NUM_CORES, num_subcores=NUM_SUBCORES)

    def body(g_hbm, ei_hbm, out_hbm, acc_sh, g_sh, srcb, dstb,
             rows0, rows1, rows2, rows3, zt, tidx_v, trows, gsem, ssem):
        c = lax.axis_index("c")
        s = lax.axis_index("s")
        w = s * NUM_CORES + c
        ebase = w * epw

        @pl.when(s < nload)
        def _():
            rows_here = min(lt, n - (nload - 1) * lt)
            lbase = s * lt

            @pl.when(s < nload - 1)
            def _():
                pltpu.sync_copy(g_hbm.at[pl.ds(lbase, lt)],
                                g_sh.at[pl.ds(lbase, lt)])

            @pl.when(s == nload - 1)
            def _():
                pltpu.sync_copy(g_hbm.at[pl.ds((nload - 1) * lt, rows_here)],
                                g_sh.at[pl.ds((nload - 1) * lt, rows_here)])

        def zb(i, carry):
            zt[i, :] = jnp.zeros((LANES,), jnp.float32)
            return carry
        lax.fori_loop(0, rows_per_tile, zb, None)
        pltpu.sync_copy(zt, acc_sh.at[pl.ds(s * rows_per_tile, rows_per_tile)])
        pltpu.sync_copy(ei_hbm.at[0].at[pl.ds(ebase, epw)], srcb)
        pltpu.sync_copy(ei_hbm.at[1].at[pl.ds(ebase, epw)], dstb)
        plsc.subcore_barrier()
        g_hbm = g_sh

        rows = (rows0, rows1, rows2, rows3)
        nbuf = len(rows)
        depth = nbuf - 1

        def sidx(j):
            return srcb.at[pl.ds(j * CHUNK, CHUNK)]

        def didx(j):
            return dstb.at[pl.ds(j * CHUNK, CHUNK)]

        for k in range(depth):
            pltpu.async_copy(g_hbm.at[sidx(k)], rows[k], gsem)

        def group(jg, carry):
            for b in range(nbuf):
                j = jg * nbuf + b
                buf = rows[b]
                pltpu.make_async_copy(g_hbm.at[sidx(j)], buf, gsem).wait()
                pltpu.async_copy(buf, acc_sh.at[didx(j)], ssem, add=True)
                prv = rows[(b - 1) % nbuf]

                @pl.when(j >= 1)
                def _():
                    pltpu.make_async_copy(prv, acc_sh.at[didx(j - 1)],
                                          ssem).wait()

                @pl.when(j + depth < fc)
                def _():
                    pltpu.async_copy(g_hbm.at[sidx(j + depth)], prv, gsem)
            return carry
        lax.fori_loop(0, fc // nbuf, group, None)
        for j in range(fc - fc % nbuf, fc):
            b = j % nbuf
            pltpu.make_async_copy(g_hbm.at[sidx(j)], rows[b], gsem).wait()
            pltpu.async_copy(rows[b], acc_sh.at[didx(j)], ssem, add=True)
            pltpu.make_async_copy(rows[(b - 1) % nbuf],
                                  acc_sh.at[didx(j - 1)], ssem).wait()
        pltpu.make_async_copy(rows[(fc - 1) % nbuf],
                              acc_sh.at[didx(fc - 1)], ssem).wait()

        if tail:
            pltpu.sync_copy(ei_hbm.at[0].at[pl.ds(ebase + fc * CHUNK, tail)],
                            tidx_v)
            pltpu.async_copy(g_hbm.at[tidx_v], trows, gsem).wait()
            pltpu.sync_copy(ei_hbm.at[1].at[pl.ds(ebase + fc * CHUNK, tail)],
                            tidx_v)
            pltpu.sync_copy(trows, acc_sh.at[tidx_v], add=True)

        plsc.subcore_barrier()
        base = s * rows_per_tile
        pltpu.sync_copy(acc_sh.at[pl.ds(base, rows_per_tile)],
                        out_hbm.at[c].at[pl.ds(base, rows_per_tile)])

    return pl.kernel(
        body,
        out_type=jax.ShapeDtypeStruct((NUM_CORES, np_, LANES), jnp.float32),
        mesh=mesh,
        compiler_params=pltpu.CompilerParams(use_tc_tiling_on_sc=False),
        scratch_types=[
            pltpu.VMEM_SHARED((np_, LANES), jnp.float32),
            pltpu.VMEM_SHARED((n, LANES), jnp.float32),
            pltpu.VMEM((epw,), jnp.int32),
            pltpu.VMEM((epw,), jnp.int32),
            pltpu.VMEM((CHUNK, LANES), jnp.float32),
            pltpu.VMEM((CHUNK, LANES), jnp.float32),
            pltpu.VMEM((CHUNK, LANES), jnp.float32),
            pltpu.VMEM((CHUNK, LANES), jnp.float32),
            pltpu.VMEM((rows_per_tile, LANES), jnp.float32),
            pltpu.VMEM((tail if tail else LANES,), jnp.int32),
            pltpu.VMEM((tail if tail else LANES, LANES), jnp.float32),
            pltpu.SemaphoreType.DMA,
            pltpu.SemaphoreType.DMA,
        ],
    )


_FW = 128
_GRP = _FW // LANES


def _tc_scale_matmul(x8, w1b, cnt_f, nf, npf):
    def body(x_ref, w_ref, c_ref, o_ref, d_ref):
        dis = jax.lax.rsqrt(c_ref[0, :nf] + c_ref[1, :nf] + 1.0)
        d_ref[...] = dis
        h = jnp.dot(x_ref[...], w_ref[...], preferred_element_type=jnp.float32)
        o_ref[...] = dis * h

    return pl.pallas_call(
        body,
        out_shape=[jax.ShapeDtypeStruct((nf, _FW), jnp.float32),
                   jax.ShapeDtypeStruct((nf, _FW), jnp.float32)],
    )(x8, w1b, cnt_f)


def _tc_mid_layer(p_f, g1_f, dis_f, b1, w2b, nf):
    def body(p_ref, g_ref, d_ref, b_ref, w_ref, o_ref):
        agg = p_ref[0, :nf] + p_ref[1, :nf] + g_ref[...]
        h = jnp.maximum(d_ref[...] * agg + b_ref[...], 0.0)
        o_ref[...] = d_ref[...] * jnp.dot(
            h, w_ref[...], preferred_element_type=jnp.float32)

    return pl.pallas_call(
        body,
        out_shape=jax.ShapeDtypeStruct((nf, _FW), jnp.float32),
    )(p_f, g1_f, dis_f, b1, w2b)


def _tc_final_layer(q_f, g2_f, dis_f, b2, nf):
    def body(q_ref, g_ref, d_ref, b_ref, o_ref):
        agg = q_ref[0, :nf] + q_ref[1, :nf] + g_ref[...]
        o_ref[...] = jax.nn.sigmoid(d_ref[...] * agg + b_ref[...])

    return pl.pallas_call(
        body,
        out_shape=jax.ShapeDtypeStruct((nf, _FW), jnp.float32),
    )(q_f, g2_f, dis_f, b2)


def kernel(x, edge_index, W1, b1, W2, b2):
    n, f0 = x.shape
    e = edge_index.shape[1]
    f1 = W1.shape[1]
    f2 = W2.shape[1]

    np_ = -(-(n + LANES) // 256) * 256
    if e % (NUM_WORKERS * LANES):
        e_pad = -(-e // (NUM_WORKERS * LANES)) * NUM_WORKERS * LANES
        padi = jnp.arange(e_pad - e, dtype=edge_index.dtype)
        ei = jnp.concatenate(
            [edge_index, jnp.stack([padi % LANES, n + (padi % LANES)])], axis=1)
    else:
        e_pad = e
        ei = edge_index
    epw = e_pad // NUM_WORKERS
    nf = n * LANES // _FW
    npf = np_ * LANES // _FW

    b1row = jnp.tile(b1, _GRP).reshape(1, _FW)
    w1b = jnp.kron(jnp.eye(_GRP, dtype=jnp.float32), W1)
    w2pad = jnp.zeros((f1, LANES), jnp.float32).at[:, :f2].set(W2)
    w2b = jnp.kron(jnp.eye(_GRP, dtype=jnp.float32), w2pad)
    b2row = jnp.tile(jnp.zeros((LANES,), jnp.float32).at[:f2].set(b2),
                     _GRP).reshape(1, _FW)

    cnt = _make_deg_kernel(np_, e_pad, epw)(ei)
    agg = _make_agg_kernel(np_, e_pad, epw, n)

    g1_f, dis_f = _tc_scale_matmul(x.reshape(nf, f0 * _GRP), w1b,
                                   cnt.reshape(2, npf, _FW), nf, npf)
    p = agg(g1_f.reshape(n, LANES), ei)
    g2_f = _tc_mid_layer(p.reshape(2, npf, _FW), g1_f, dis_f, b1row, w2b, nf)
    q = agg(g2_f.reshape(n, LANES), ei)
    out_f = _tc_final_layer(q.reshape(2, npf, _FW), g2_f, dis_f, b2row, nf)
    return out_f.reshape(n, LANES)[:, :f2]
